# baseline (speedup 1.0000x reference)
import functools

import jax
import jax.numpy as jnp
from jax import lax
from jax.experimental import pallas as pl
from jax.experimental.pallas import tpu as pltpu
from jax.experimental.pallas import tpu_sc as plsc


def kernel(x, W, H):
    B = x.shape[0]
    D = W.shape[1]
    info = plsc.get_sparse_core_info()
    NC, NS, L = info.num_cores, info.num_subcores, info.num_lanes
    NW = NC * NS
    b_per_w = B // NW
    CH = 128
    n_ch = b_per_w // CH

    u_idx = x[:, 0].reshape(NW, n_ch, CH)
    v_idx = x[:, 1].reshape(NW, n_ch, CH)

    mesh = plsc.VectorSubcoreMesh(core_axis_name="c", subcore_axis_name="s")

    @functools.partial(
        pl.kernel,
        out_type=jax.ShapeDtypeStruct((NW, b_per_w), jnp.float32),
        mesh=mesh,
        compiler_params=pltpu.CompilerParams(
            needs_layout_passes=False, use_tc_tiling_on_sc=False),
        scratch_types=[
            pltpu.VMEM((n_ch, CH), jnp.int32),
            pltpu.VMEM((n_ch, CH), jnp.int32),
            pltpu.VMEM((b_per_w, D), jnp.float32),
            pltpu.VMEM((b_per_w, D), jnp.float32),
            pltpu.VMEM((b_per_w,), jnp.float32),
            pltpu.SemaphoreType.DMA,
        ],
    )
    def sc_kernel(uidx_hbm, vidx_hbm, w_hbm, h_hbm, out_hbm,
                  uidx_v, vidx_v, urows, vrows, outv, sem):
        wid = lax.axis_index("s") * NC + lax.axis_index("c")

        pltpu.sync_copy(uidx_hbm.at[wid], uidx_v)
        pltpu.sync_copy(vidx_hbm.at[wid], vidx_v)

        copies = []
        for j in range(n_ch):
            copies.append(pltpu.async_copy(
                w_hbm.at[uidx_v.at[j]], urows.at[pl.ds(j * CH, CH)], sem))
            copies.append(pltpu.async_copy(
                h_hbm.at[vidx_v.at[j]], vrows.at[pl.ds(j * CH, CH)], sem))
        for c in copies:
            c.wait()

        iota = lax.iota(jnp.int32, L)

        def group_body(g, carry):
            rows = g * L + iota
            accs = [jnp.zeros((L,), jnp.float32) for _ in range(4)]
            for d in range(D):
                cols = jnp.full((L,), d, jnp.int32)
                u = plsc.load_gather(urows, [rows, cols])
                v = plsc.load_gather(vrows, [rows, cols])
                accs[d % 4] = accs[d % 4] + u * v
            outv[pl.ds(g * L, L)] = (accs[0] + accs[1]) + (accs[2] + accs[3])
            return carry

        lax.fori_loop(0, b_per_w // L, group_body, 0)

        pltpu.sync_copy(outv, out_hbm.at[wid])

    out = sc_kernel(u_idx, v_idx, W, H)
    return out.reshape(B)
